# baseline (speedup 1.0000x reference)
import functools
import jax
import jax.numpy as jnp
from jax import lax
from jax.experimental import pallas as pl
from jax.experimental.pallas import tpu as pltpu
from jax.experimental.pallas import tpu_sc as plsc

NC = 2
NS = 16
NW = NC * NS
L = 16
H = 2


def _pick_div(n, limit, mult):
    for c in range(limit, 0, -1):
        if c % mult == 0 and n % c == 0:
            return c
    raise ValueError((n, limit, mult))


def _sc_mesh():
    return plsc.VectorSubcoreMesh(
        core_axis_name="c", subcore_axis_name="s",
        num_cores=NC, num_subcores=NS)


def _proj_body(x_ref, wq, bq, wk, bk, wv, bv, ws, bs, q_o, k_o, v_o, s_o):
    xb = x_ref[...]
    c = q_o.shape[2]
    q = jnp.dot(xb, wq[...], preferred_element_type=jnp.float32) + bq[...][None, :]
    k = jnp.dot(xb, wk[...], preferred_element_type=jnp.float32) + bk[...][None, :]
    v = jnp.dot(xb, wv[...], preferred_element_type=jnp.float32) + bv[...][None, :]
    for h in range(H):
        q_o[h] = q[:, h * c:(h + 1) * c]
        k_o[h] = k[:, h * c:(h + 1) * c]
        v_o[h] = v[:, h * c:(h + 1) * c]
    s_o[...] = jnp.dot(xb, ws[...], preferred_element_type=jnp.float32) + bs[...][None, :]


def _projections(x, WqT, bq, WkT, bk, WvT, bv, WsT, bs):
    n, d = x.shape
    hc = WqT.shape[1]
    c = hc // H
    bn = _pick_div(n, 1024, 8)
    wspec = pl.BlockSpec((d, hc), lambda i: (0, 0))
    bspec = pl.BlockSpec((hc,), lambda i: (0,))
    rowspec = pl.BlockSpec((bn, d), lambda i: (i, 0))
    hspec = pl.BlockSpec((H, bn, c), lambda i: (0, i, 0))
    sspec = pl.BlockSpec((bn, hc), lambda i: (i, 0))
    hshape = jax.ShapeDtypeStruct((H, n, c), jnp.float32)
    return pl.pallas_call(
        _proj_body,
        grid=(n // bn,),
        in_specs=[rowspec, wspec, bspec, wspec, bspec, wspec, bspec, wspec, bspec],
        out_specs=[hspec, hspec, hspec, sspec],
        out_shape=[hshape, hshape, hshape,
                   jax.ShapeDtypeStruct((n, hc), jnp.float32)],
    )(x, WqT, bq, WkT, bk, WvT, bv, WsT, bs)


def _rel_time(last_update, src, t):
    e = src.shape[0]
    n = last_update.shape[0]
    epw = e // NW

    @functools.partial(
        pl.kernel,
        out_type=jax.ShapeDtypeStruct((e,), jnp.float32),
        mesh=_sc_mesh(),
        compiler_params=pltpu.CompilerParams(needs_layout_passes=False),
        scratch_types=[
            pltpu.VMEM((n,), jnp.int32),
            pltpu.VMEM((epw,), jnp.int32),
            pltpu.VMEM((epw,), jnp.int32),
            pltpu.VMEM((epw,), jnp.float32),
        ],
    )
    def relk(lu_hbm, src_hbm, t_hbm, rel_hbm, lub, srcb, tb, relb):
        cid = lax.axis_index("c")
        sid = lax.axis_index("s")
        base = (sid * NC + cid) * epw
        pltpu.sync_copy(lu_hbm, lub)
        pltpu.sync_copy(src_hbm.at[pl.ds(base, epw)], srcb)
        pltpu.sync_copy(t_hbm.at[pl.ds(base, epw)], tb)

        def g(j, c):
            sl = pl.ds(j * L, L)
            lu16 = plsc.load_gather(lub, [srcb[sl]])
            relb[sl] = (lu16 - tb[sl]).astype(jnp.float32)
            return c

        lax.fori_loop(0, epw // L, g, 0)
        pltpu.sync_copy(relb, rel_hbm.at[pl.ds(base, epw)])

    return relk(last_update, src, t)


def _edge_feat_body(rel_ref, msg_ref, wv_ref, bv_ref, wt_ref, wm_ref, be_ref, e_o):
    rel = rel_ref[0, 0, :]
    c = e_o.shape[2]
    tf = jnp.cos(rel[:, None] * wv_ref[...][None, :] + bv_ref[...][None, :])
    acc = jnp.dot(tf, wt_ref[...], preferred_element_type=jnp.float32)
    acc = acc + jnp.dot(msg_ref[...], wm_ref[...], preferred_element_type=jnp.float32)
    acc = acc + be_ref[...][None, :]
    for h in range(H):
        e_o[h] = acc[:, h * c:(h + 1) * c]


def _edge_feat(rel, msg, wvec, bvec, WtT, WmT, be):
    e = rel.shape[0]
    td = wvec.shape[0]
    m = msg.shape[1]
    hc = WtT.shape[1]
    c = hc // H
    blk = _pick_div(e, 2560, 128)
    full = lambda shape: pl.BlockSpec(shape, lambda i: tuple(0 for _ in shape))
    return pl.pallas_call(
        _edge_feat_body,
        grid=(e // blk,),
        in_specs=[
            pl.BlockSpec((1, 1, blk), lambda i: (i, 0, 0)),
            pl.BlockSpec((blk, m), lambda i: (i, 0)),
            full((td,)),
            full((td,)),
            full((td, hc)),
            full((m, hc)),
            full((hc,)),
        ],
        out_specs=pl.BlockSpec((H, blk, c), lambda i: (0, i, 0)),
        out_shape=jax.ShapeDtypeStruct((H, e, c), jnp.float32),
    )(rel.reshape(e // blk, 1, blk), msg, wvec, bvec, WtT, WmT, be)


def _edge_attn(qh, kh, vh, eh, src, dst, scale):
    _, n, c = qh.shape
    e = src.shape[0]
    epc = e // NS
    ch = _pick_div(epc, 80, 8)
    nch = epc // ch
    z = _pick_div(n, 80, 8)
    nchk = n // z
    iters = -(-nchk // NS)
    nreg = c // L
    qf = qh.reshape(H * n, c)
    kf = kh.reshape(H * n, c)
    vf = vh.reshape(H * n, c)
    src3 = src.reshape(NS, nch, ch)
    dst3 = dst.reshape(NS, nch, ch)

    @functools.partial(
        pl.kernel,
        out_type=[
            jax.ShapeDtypeStruct((H, n, c), jnp.float32),
            jax.ShapeDtypeStruct((H, n, L), jnp.float32),
        ],
        mesh=_sc_mesh(),
        compiler_params=pltpu.CompilerParams(
            needs_layout_passes=False, use_tc_tiling_on_sc=False),
        scratch_types=[
            pltpu.VMEM_SHARED((n, c), jnp.float32),
            pltpu.VMEM_SHARED((n, L), jnp.float32),
            pltpu.VMEM((4, ch), jnp.int32),
            pltpu.VMEM((4, ch), jnp.int32),
            pltpu.VMEM((4, ch), jnp.int32),
            pltpu.VMEM((2, ch, c), jnp.float32),
            pltpu.VMEM((2, ch, c), jnp.float32),
            pltpu.VMEM((2, ch, c), jnp.float32),
            pltpu.VMEM((2, ch, c), jnp.float32),
            pltpu.VMEM((2, ch, c), jnp.float32),
            pltpu.VMEM((2, ch, L), jnp.float32),
            pltpu.VMEM((z, c), jnp.float32),
            pltpu.VMEM((z, L), jnp.float32),
            pltpu.SemaphoreType.DMA,
            pltpu.SemaphoreType.DMA,
            pltpu.SemaphoreType.DMA,
            pltpu.SemaphoreType.DMA,
        ],
    )
    def attnk(q_hbm, k_hbm, v_hbm, e_hbm, src_hbm, dst_hbm,
              aggr_hbm, asum_hbm,
              aggr_s, asum_s,
              srcb, dstb, dsta, qb, kb, vb, eb, ob, ub, zb, zb16,
              gs0, gs1, ss0, ss1):
        cid = lax.axis_index("c")
        sid = lax.axis_index("s")
        zero = jnp.zeros((L,), jnp.float32)
        noff = cid * n
        gsem = (gs0, gs1)
        ssem = (ss0, ss1)

        def zrow(j, cc):
            for r in range(nreg):
                zb[j, pl.ds(r * L, L)] = zero
            zb16[j, :] = zero
            return cc

        lax.fori_loop(0, z, zrow, 0)
        for m in range(iters):
            cidx = m * NS + sid

            @pl.when(cidx < nchk)
            def _():
                base = cidx * z
                pltpu.sync_copy(zb, aggr_s.at[pl.ds(base, z)])
                pltpu.sync_copy(zb16, asum_s.at[pl.ds(base, z)])

        plsc.subcore_barrier()

        ebase = sid * epc

        def load_idx(g):
            r = g % 4
            pltpu.sync_copy(src_hbm.at[sid, g], srcb.at[r])
            pltpu.sync_copy(dst_hbm.at[sid, g], dstb.at[r])
            for i in range(ch // L):
                sl = pl.ds(i * L, L)
                srcb[r, sl] = srcb[r, sl] + noff
                dsta[r, sl] = dstb[r, sl] + noff

        def gather_descs(g, par):
            r = g % 4
            return [
                pltpu.make_async_copy(q_hbm.at[dsta.at[r]], qb.at[par], gsem[par]),
                pltpu.make_async_copy(k_hbm.at[srcb.at[r]], kb.at[par], gsem[par]),
                pltpu.make_async_copy(v_hbm.at[srcb.at[r]], vb.at[par], gsem[par]),
                pltpu.make_async_copy(
                    e_hbm.at[cid, pl.ds(ebase + g * ch, ch)], eb.at[par],
                    gsem[par]),
            ]

        def start_scatters(g, par):
            r = g % 4
            pltpu.async_copy(ob.at[par], aggr_s.at[dstb.at[r]],
                             ssem[par], add=True)
            pltpu.async_copy(ub.at[par], asum_s.at[dstb.at[r]],
                             ssem[par], add=True)

        def wait_scatters(g, par):
            r = g % 4
            pltpu.make_async_copy(ob.at[par], aggr_s.at[dstb.at[r]],
                                  ssem[par]).wait()
            pltpu.make_async_copy(ub.at[par], asum_s.at[dstb.at[r]],
                                  ssem[par]).wait()

        load_idx(0)
        for d in gather_descs(0, 0):
            d.start()

        def pair(gi, cc):
            for par in range(2):
                g = gi * 2 + par
                nxt = 1 - par

                @pl.when(g + 1 < nch)
                def _():
                    load_idx(g + 1)
                    for d in gather_descs(g + 1, nxt):
                        d.start()

                @pl.when(g >= 2)
                def _():
                    wait_scatters(g - 2, par)

                for d in gather_descs(g, par):
                    d.wait()

                qp = qb.at[par]
                kp = kb.at[par]
                vp = vb.at[par]
                ep = eb.at[par]
                op = ob.at[par]
                up = ub.at[par]

                def edge(j, cc2):
                    acc = None
                    for r in range(nreg):
                        sl = pl.ds(r * L, L)
                        p = qp[j, sl] * (kp[j, sl] + ep[j, sl])
                        acc = p if acc is None else acc + p
                    s = jnp.sum(acc) * scale
                    u = jnp.exp(jnp.broadcast_to(s, (L,)))
                    for r in range(nreg):
                        sl = pl.ds(r * L, L)
                        op[j, sl] = u * (vp[j, sl] + ep[j, sl])
                    lanes = lax.iota(jnp.int32, L)
                    up[j, :] = jnp.where(lanes == 0, u, zero)
                    return cc2

                lax.fori_loop(0, ch, edge, 0)
                start_scatters(g, par)
            return cc

        lax.fori_loop(0, nch // 2, pair, 0)
        for g in (nch - 2, nch - 1):
            wait_scatters(g, g % 2)
        plsc.subcore_barrier()

        for m in range(iters):
            cidx = m * NS + sid

            @pl.when(cidx < nchk)
            def _():
                base = cidx * z
                pltpu.sync_copy(aggr_s.at[pl.ds(base, z)],
                                aggr_hbm.at[cid, pl.ds(base, z)])
                pltpu.sync_copy(asum_s.at[pl.ds(base, z)],
                                asum_hbm.at[cid, pl.ds(base, z)])

    return attnk(qf, kf, vf, eh, src3, dst3)


def _final_body(aggr_ref, asum_ref, skip_ref, o_ref):
    bn, c = aggr_ref.shape[1:]
    parts = []
    for h in range(H):
        a = jnp.broadcast_to(asum_ref[h, :, 0:1], (bn, c))
        parts.append(aggr_ref[h] / (a + 1e-16))
    o_ref[...] = jnp.concatenate(parts, axis=1) + skip_ref[...]


def _finalize(aggr, asum, skip):
    n, hc = skip.shape
    c = hc // H
    bn = _pick_div(n, 1024, 8)
    return pl.pallas_call(
        _final_body,
        grid=(n // bn,),
        in_specs=[
            pl.BlockSpec((H, bn, c), lambda i: (0, i, 0)),
            pl.BlockSpec((H, bn, L), lambda i: (0, i, 0)),
            pl.BlockSpec((bn, hc), lambda i: (i, 0)),
        ],
        out_specs=pl.BlockSpec((bn, hc), lambda i: (i, 0)),
        out_shape=jax.ShapeDtypeStruct((n, hc), jnp.float32),
    )(aggr, asum, skip)


def kernel(x, last_update, edge_index, t, msg, W_time, b_time,
           Wq, bq, Wk, bk, Wv, bv, We, be, Ws, bs):
    src = edge_index[0]
    dst = edge_index[1]
    td = W_time.shape[0]
    c = Wq.shape[0] // H
    scale = 1.0 / float(c) ** 0.5

    qh, kh, vh, skip = _projections(x, Wq.T, bq, Wk.T, bk, Wv.T, bv, Ws.T, bs)
    rel = _rel_time(last_update, src, t)
    WeT = We.T
    eh = _edge_feat(rel, msg, W_time[:, 0], b_time, WeT[:td], WeT[td:], be)
    aggr, asum = _edge_attn(qh, kh, vh, eh, src, dst, scale)
    return _finalize(aggr, asum, skip)


# speedup vs baseline: 1.2367x; 1.2367x over previous
import functools
import jax
import jax.numpy as jnp
from jax import lax
from jax.experimental import pallas as pl
from jax.experimental.pallas import tpu as pltpu
from jax.experimental.pallas import tpu_sc as plsc

NC = 2
NS = 16
NW = NC * NS
L = 16
H = 2


def _pick_div(n, limit, mult):
    for c in range(limit, 0, -1):
        if c % mult == 0 and n % c == 0:
            return c
    raise ValueError((n, limit, mult))


def _sc_mesh():
    return plsc.VectorSubcoreMesh(
        core_axis_name="c", subcore_axis_name="s",
        num_cores=NC, num_subcores=NS)


def _proj_body(x_ref, wq, bq, wk, bk, wv, bv, ws, bs, q_o, k_o, v_o, s_o):
    xb = x_ref[...]
    c = q_o.shape[2]
    q = jnp.dot(xb, wq[...], preferred_element_type=jnp.float32) + bq[...][None, :]
    k = jnp.dot(xb, wk[...], preferred_element_type=jnp.float32) + bk[...][None, :]
    v = jnp.dot(xb, wv[...], preferred_element_type=jnp.float32) + bv[...][None, :]
    for h in range(H):
        q_o[h] = q[:, h * c:(h + 1) * c]
        k_o[h] = k[:, h * c:(h + 1) * c]
        v_o[h] = v[:, h * c:(h + 1) * c]
    s_o[...] = jnp.dot(xb, ws[...], preferred_element_type=jnp.float32) + bs[...][None, :]


def _projections(x, WqT, bq, WkT, bk, WvT, bv, WsT, bs):
    n, d = x.shape
    hc = WqT.shape[1]
    c = hc // H
    bn = _pick_div(n, 1024, 8)
    wspec = pl.BlockSpec((d, hc), lambda i: (0, 0))
    bspec = pl.BlockSpec((hc,), lambda i: (0,))
    rowspec = pl.BlockSpec((bn, d), lambda i: (i, 0))
    hspec = pl.BlockSpec((H, bn, c), lambda i: (0, i, 0))
    sspec = pl.BlockSpec((bn, hc), lambda i: (i, 0))
    hshape = jax.ShapeDtypeStruct((H, n, c), jnp.float32)
    return pl.pallas_call(
        _proj_body,
        grid=(n // bn,),
        in_specs=[rowspec, wspec, bspec, wspec, bspec, wspec, bspec, wspec, bspec],
        out_specs=[hspec, hspec, hspec, sspec],
        out_shape=[hshape, hshape, hshape,
                   jax.ShapeDtypeStruct((n, hc), jnp.float32)],
    )(x, WqT, bq, WkT, bk, WvT, bv, WsT, bs)


def _rel_time(last_update, src, t):
    e = src.shape[0]
    n = last_update.shape[0]
    epw = e // NW

    @functools.partial(
        pl.kernel,
        out_type=jax.ShapeDtypeStruct((e,), jnp.float32),
        mesh=_sc_mesh(),
        compiler_params=pltpu.CompilerParams(needs_layout_passes=False),
        scratch_types=[
            pltpu.VMEM((n,), jnp.int32),
            pltpu.VMEM((epw,), jnp.int32),
            pltpu.VMEM((epw,), jnp.int32),
            pltpu.VMEM((epw,), jnp.float32),
        ],
    )
    def relk(lu_hbm, src_hbm, t_hbm, rel_hbm, lub, srcb, tb, relb):
        cid = lax.axis_index("c")
        sid = lax.axis_index("s")
        base = (sid * NC + cid) * epw
        pltpu.sync_copy(lu_hbm, lub)
        pltpu.sync_copy(src_hbm.at[pl.ds(base, epw)], srcb)
        pltpu.sync_copy(t_hbm.at[pl.ds(base, epw)], tb)

        def g(j, c):
            sl = pl.ds(j * L, L)
            lu16 = plsc.load_gather(lub, [srcb[sl]])
            relb[sl] = (lu16 - tb[sl]).astype(jnp.float32)
            return c

        lax.fori_loop(0, epw // L, g, 0)
        pltpu.sync_copy(relb, rel_hbm.at[pl.ds(base, epw)])

    return relk(last_update, src, t)


def _edge_feat_body(rel_ref, msg_ref, wv_ref, bv_ref, wt_ref, wm_ref, be_ref, e_o):
    rel = rel_ref[0, 0, :]
    tf = jnp.cos(rel[:, None] * wv_ref[...][None, :] + bv_ref[...][None, :])
    acc = jnp.dot(tf, wt_ref[...], preferred_element_type=jnp.float32)
    acc = acc + jnp.dot(msg_ref[...], wm_ref[...], preferred_element_type=jnp.float32)
    acc = acc + be_ref[...][None, :]
    c = e_o.shape[2]
    for h in range(H):
        e_o[h] = acc[:, h * c:(h + 1) * c]


def _edge_feat(rel, msg, wvec, bvec, WtT, WmT, be):
    e = rel.shape[0]
    td = wvec.shape[0]
    m = msg.shape[1]
    hc = WtT.shape[1]
    c = hc // H
    blk = _pick_div(e, 2560, 128)
    full = lambda shape: pl.BlockSpec(shape, lambda i: tuple(0 for _ in shape))
    return pl.pallas_call(
        _edge_feat_body,
        grid=(e // blk,),
        in_specs=[
            pl.BlockSpec((1, 1, blk), lambda i: (i, 0, 0)),
            pl.BlockSpec((blk, m), lambda i: (i, 0)),
            full((td,)),
            full((td,)),
            full((td, hc)),
            full((m, hc)),
            full((hc,)),
        ],
        out_specs=pl.BlockSpec((H, blk, c), lambda i: (0, i, 0)),
        out_shape=jax.ShapeDtypeStruct((H, e, c), jnp.float32),
    )(rel.reshape(e // blk, 1, blk), msg, wvec, bvec, WtT, WmT, be)


def _edge_attn(qh, kh, vh, eh, src, dst, scale, init=None):
    _, n, c = qh.shape
    e = src.shape[0]
    has_init = init is not None
    epc = e // NS
    ch = _pick_div(epc, 80, 8)
    nch = epc // ch
    z = _pick_div(n, 80, 8)
    nchk = n // z
    iters = -(-nchk // NS)
    nreg = c // L
    qf = qh.reshape(H * n, c)
    kf = kh.reshape(H * n, c)
    vf = vh.reshape(H * n, c)
    src3 = src.reshape(NS, nch, ch)
    dst3 = dst.reshape(NS, nch, ch)

    @functools.partial(
        pl.kernel,
        out_type=[
            jax.ShapeDtypeStruct((H, n, c), jnp.float32),
            jax.ShapeDtypeStruct((H, n, L), jnp.float32),
        ],
        mesh=_sc_mesh(),
        compiler_params=pltpu.CompilerParams(
            needs_layout_passes=False, use_tc_tiling_on_sc=False),
        scratch_types=[
            pltpu.VMEM_SHARED((n, c), jnp.float32),
            pltpu.VMEM_SHARED((n, L), jnp.float32),
            pltpu.VMEM((4, ch), jnp.int32),
            pltpu.VMEM((4, ch), jnp.int32),
            pltpu.VMEM((4, ch), jnp.int32),
            pltpu.VMEM((2, ch, c), jnp.float32),
            pltpu.VMEM((2, ch, c), jnp.float32),
            pltpu.VMEM((2, ch, c), jnp.float32),
            pltpu.VMEM((2, ch, c), jnp.float32),
            pltpu.VMEM((2, ch, c), jnp.float32),
            pltpu.VMEM((2, ch, L), jnp.float32),
            pltpu.VMEM((z, c), jnp.float32),
            pltpu.VMEM((z, L), jnp.float32),
            pltpu.SemaphoreType.DMA,
            pltpu.SemaphoreType.DMA,
            pltpu.SemaphoreType.DMA,
            pltpu.SemaphoreType.DMA,
        ],
    )
    def attnk(q_hbm, k_hbm, v_hbm, e_hbm, src_hbm, dst_hbm, *rest):
        if has_init:
            (ia_hbm, iu_hbm, aggr_hbm, asum_hbm,
             aggr_s, asum_s,
             srcb, dstb, dsta, qb, kb, vb, eb, ob, ub, zb, zb16,
             gs0, gs1, ss0, ss1) = rest
        else:
            (aggr_hbm, asum_hbm,
             aggr_s, asum_s,
             srcb, dstb, dsta, qb, kb, vb, eb, ob, ub, zb, zb16,
             gs0, gs1, ss0, ss1) = rest
        cid = lax.axis_index("c")
        sid = lax.axis_index("s")
        zero = jnp.zeros((L,), jnp.float32)
        noff = cid * n
        gsem = (gs0, gs1)
        ssem = (ss0, ss1)

        if not has_init:
            def zrow(j, cc):
                for r in range(nreg):
                    zb[j, pl.ds(r * L, L)] = zero
                zb16[j, :] = zero
                return cc

            lax.fori_loop(0, z, zrow, 0)
        for m in range(iters):
            cidx = m * NS + sid

            @pl.when(cidx < nchk)
            def _():
                base = cidx * z
                sl = pl.ds(base, z)
                if has_init:
                    pltpu.sync_copy(ia_hbm.at[cid, sl], aggr_s.at[sl])
                    pltpu.sync_copy(iu_hbm.at[cid, sl], asum_s.at[sl])
                else:
                    pltpu.sync_copy(zb, aggr_s.at[sl])
                    pltpu.sync_copy(zb16, asum_s.at[sl])

        plsc.subcore_barrier()

        ebase = sid * epc

        def load_idx(g):
            r = g % 4
            pltpu.sync_copy(src_hbm.at[sid, g], srcb.at[r])
            pltpu.sync_copy(dst_hbm.at[sid, g], dstb.at[r])
            for i in range(ch // L):
                sl = pl.ds(i * L, L)
                srcb[r, sl] = srcb[r, sl] + noff
                dsta[r, sl] = dstb[r, sl] + noff

        def gather_descs(g, par):
            r = g % 4
            return [
                pltpu.make_async_copy(q_hbm.at[dsta.at[r]], qb.at[par], gsem[par]),
                pltpu.make_async_copy(k_hbm.at[srcb.at[r]], kb.at[par], gsem[par]),
                pltpu.make_async_copy(v_hbm.at[srcb.at[r]], vb.at[par], gsem[par]),
                pltpu.make_async_copy(
                    e_hbm.at[cid, pl.ds(ebase + g * ch, ch)], eb.at[par],
                    gsem[par]),
            ]

        def start_scatters(g, par):
            r = g % 4
            pltpu.async_copy(ob.at[par], aggr_s.at[dstb.at[r]],
                             ssem[par], add=True)
            pltpu.async_copy(ub.at[par], asum_s.at[dstb.at[r]],
                             ssem[par], add=True)

        def wait_scatters(g, par):
            r = g % 4
            pltpu.make_async_copy(ob.at[par], aggr_s.at[dstb.at[r]],
                                  ssem[par]).wait()
            pltpu.make_async_copy(ub.at[par], asum_s.at[dstb.at[r]],
                                  ssem[par]).wait()

        load_idx(0)
        for d in gather_descs(0, 0):
            d.start()

        def pair(gi, cc):
            for par in range(2):
                g = gi * 2 + par
                nxt = 1 - par

                @pl.when(g + 1 < nch)
                def _():
                    load_idx(g + 1)
                    for d in gather_descs(g + 1, nxt):
                        d.start()

                @pl.when(g >= 2)
                def _():
                    wait_scatters(g - 2, par)

                for d in gather_descs(g, par):
                    d.wait()

                qp = qb.at[par]
                kp = kb.at[par]
                vp = vb.at[par]
                ep = eb.at[par]
                op = ob.at[par]
                up = ub.at[par]

                def edge(j, cc2):
                    acc = None
                    for r in range(nreg):
                        sl = pl.ds(r * L, L)
                        p = qp[j, sl] * (kp[j, sl] + ep[j, sl])
                        acc = p if acc is None else acc + p
                    s = jnp.sum(acc) * scale
                    u = jnp.exp(jnp.broadcast_to(s, (L,)))
                    for r in range(nreg):
                        sl = pl.ds(r * L, L)
                        op[j, sl] = u * (vp[j, sl] + ep[j, sl])
                    lanes = lax.iota(jnp.int32, L)
                    up[j, :] = jnp.where(lanes == 0, u, zero)
                    return cc2

                lax.fori_loop(0, ch, edge, 0)
                start_scatters(g, par)
            return cc

        lax.fori_loop(0, nch // 2, pair, 0)
        for g in (nch - 2, nch - 1):
            wait_scatters(g, g % 2)
        plsc.subcore_barrier()

        for m in range(iters):
            cidx = m * NS + sid

            @pl.when(cidx < nchk)
            def _():
                base = cidx * z
                pltpu.sync_copy(aggr_s.at[pl.ds(base, z)],
                                aggr_hbm.at[cid, pl.ds(base, z)])
                pltpu.sync_copy(asum_s.at[pl.ds(base, z)],
                                asum_hbm.at[cid, pl.ds(base, z)])

    if has_init:
        return attnk(qf, kf, vf, eh, src3, dst3, init[0], init[1])
    return attnk(qf, kf, vf, eh, src3, dst3)


def _final_body(*refs):
    parts, skip_ref, o_ref = refs[:-2], refs[-2], refs[-1]
    ns = len(parts) // 2
    aggrs, asums = parts[:ns], parts[ns:]
    bn, c = aggrs[0].shape[1:]
    cols = []
    for h in range(H):
        aggr = aggrs[0][h]
        asum = asums[0][h, :, 0:1]
        for s in range(1, ns):
            aggr = aggr + aggrs[s][h]
            asum = asum + asums[s][h, :, 0:1]
        a = jnp.broadcast_to(asum, (bn, c))
        cols.append(aggr / (a + 1e-16))
    o_ref[...] = jnp.concatenate(cols, axis=1) + skip_ref[...]


def _finalize(aggrs, asums, skip):
    n, hc = skip.shape
    c = hc // H
    bn = _pick_div(n, 1024, 8)
    aspec = pl.BlockSpec((H, bn, c), lambda i: (0, i, 0))
    uspec = pl.BlockSpec((H, bn, L), lambda i: (0, i, 0))
    return pl.pallas_call(
        _final_body,
        grid=(n // bn,),
        in_specs=[aspec] * len(aggrs) + [uspec] * len(asums)
        + [pl.BlockSpec((bn, hc), lambda i: (i, 0))],
        out_specs=pl.BlockSpec((bn, hc), lambda i: (i, 0)),
        out_shape=jax.ShapeDtypeStruct((n, hc), jnp.float32),
    )(*aggrs, *asums, skip)


def kernel(x, last_update, edge_index, t, msg, W_time, b_time,
           Wq, bq, Wk, bk, Wv, bv, We, be, Ws, bs):
    src = edge_index[0]
    dst = edge_index[1]
    td = W_time.shape[0]
    c = Wq.shape[0] // H
    scale = 1.0 / float(c) ** 0.5

    qh, kh, vh, skip = _projections(x, Wq.T, bq, Wk.T, bk, Wv.T, bv, Ws.T, bs)
    rel = _rel_time(last_update, src, t)
    WeT = We.T

    ne = src.shape[0]
    unit = NS * 2 * 80
    nu = ne // unit
    if nu >= 2 and ne % unit == 0:
        bounds = [0, (nu // 2) * unit, ne]
    else:
        bounds = [0, ne]

    part = None
    for s0, s1 in zip(bounds[:-1], bounds[1:]):
        eh = _edge_feat(rel[s0:s1], msg[s0:s1], W_time[:, 0], b_time,
                        WeT[:td], WeT[td:], be)
        part = _edge_attn(qh, kh, vh, eh, src[s0:s1], dst[s0:s1], scale,
                          init=part)
    return _finalize([part[0]], [part[1]], skip)
